# baseline (speedup 1.0000x reference)
import jax
import jax.numpy as jnp
from jax.experimental import pallas as pl
from jax.experimental.pallas import tpu as pltpu

_NUM_BINS = 100000
_SALT_ADD = (42 * 0x9E3779B9) & 0xFFFFFFFF


def _hash_block(x_ref, o_ref):
    z = x_ref[...].astype(jnp.uint32)
    z = z + jnp.uint32(_SALT_ADD)
    z = (z ^ (z >> 16)) * jnp.uint32(0x85EBCA6B)
    z = (z ^ (z >> 13)) * jnp.uint32(0xC2B2AE35)
    z = z ^ (z >> 16)
    q = z // jnp.uint32(_NUM_BINS)
    r = z - q * jnp.uint32(_NUM_BINS)
    o_ref[...] = r.astype(jnp.int32)


def kernel(inputs):
    n, m = inputs.shape
    xt = jnp.swapaxes(inputs, 0, 1)
    xt = pltpu.with_memory_space_constraint(xt, pltpu.MemorySpace.HBM)
    gr, gc = 5, 2
    br, bc = m // gr, n // gc
    out_t = pl.pallas_call(
        _hash_block,
        grid=(gr, gc),
        in_specs=[pl.BlockSpec((br, bc), lambda i, j: (i, j))],
        out_specs=pl.BlockSpec((br, bc), lambda i, j: (i, j)),
        out_shape=jax.ShapeDtypeStruct((m, n), jnp.int32),
    )(xt)
    return jnp.swapaxes(out_t, 0, 1)
